# baseline (speedup 1.0000x reference)
import functools
import math

import jax
import jax.numpy as jnp
from jax import lax
from jax.experimental import pallas as pl
from jax.experimental.pallas import tpu as pltpu
from jax.experimental.pallas import tpu_sc as plsc

def _sc_gather(table, idx2):
    nrows = idx2.shape[0]
    d = table.shape[1]
    nw = 32
    nch = nrows // (2 * nw)
    chunk = idx2.shape[1]
    info = plsc.get_sparse_core_info()
    ncores = info.num_cores

    mesh = plsc.VectorSubcoreMesh(core_axis_name="c", subcore_axis_name="s")

    @functools.partial(
        pl.kernel,
        mesh=mesh,
        compiler_params=pltpu.CompilerParams(use_tc_tiling_on_sc=False),
        out_type=jax.ShapeDtypeStruct((nw, nch * chunk, 2 * d), jnp.float32),
        scratch_types=[
            pltpu.VMEM((nch, chunk), jnp.int32),
            pltpu.VMEM((nch, chunk), jnp.int32),
            pltpu.VMEM((chunk, d), jnp.float32),
            pltpu.VMEM((chunk, d), jnp.float32),
            pltpu.VMEM((chunk, d), jnp.float32),
            pltpu.VMEM((chunk, d), jnp.float32),
            pltpu.VMEM((chunk, 2 * d), jnp.float32),
            pltpu.VMEM((chunk, 2 * d), jnp.float32),
            pltpu.SemaphoreType.DMA,
            pltpu.SemaphoreType.DMA,
            pltpu.SemaphoreType.DMA,
            pltpu.SemaphoreType.DMA,
            pltpu.SemaphoreType.DMA,
            pltpu.SemaphoreType.DMA,
            pltpu.SemaphoreType.DMA,
        ],
    )
    def k(table_hbm, idx_hbm, out_hbm, idx_va, idx_vb, a0, a1, b0, b1, p0, p1,
          gsa0, gsa1, gsb0, gsb1, os0, os1, isem):
        wid = lax.axis_index("s") * ncores + lax.axis_index("c")

        pltpu.make_async_copy(
            idx_hbm.at[pl.ds(wid * nch, nch)], idx_va, isem).start()
        pltpu.make_async_copy(
            idx_hbm.at[pl.ds(nrows // 2 + wid * nch, nch)], idx_vb, isem).start()
        pltpu.make_async_copy(
            idx_hbm.at[pl.ds(wid * nch, nch)], idx_va, isem).wait()
        pltpu.make_async_copy(
            idx_hbm.at[pl.ds(wid * nch, nch)], idx_vb, isem).wait()

        def gather(idx_v, c, buf, sem):
            return pltpu.make_async_copy(
                table_hbm.at[idx_v.at[c]], buf, sem)

        def out_copy(c, pbuf, sem):
            return pltpu.make_async_copy(
                pbuf, out_hbm.at[wid, pl.ds(c * chunk, chunk)], sem)

        def pairup(abuf, bbuf, pbuf):
            nu = d // 16

            def body(r, carry):
                va = [abuf[r, pl.ds(16 * u, 16)] for u in range(nu)]
                vb = [bbuf[r, pl.ds(16 * u, 16)] for u in range(nu)]
                for u in range(nu):
                    pbuf[r, pl.ds(16 * u, 16)] = va[u]
                for u in range(nu):
                    pbuf[r, pl.ds(d + 16 * u, 16)] = vb[u]
                return carry
            lax.fori_loop(0, chunk, body, 0)

        gather(idx_va, 0, a0, gsa0).start()
        gather(idx_vb, 0, b0, gsb0).start()

        def step(c, abuf, gsa, bbuf, gsb, pbuf, osem, nabuf, ngsa, nbbuf, ngsb):
            @pl.when(c + 1 < nch)
            def _():
                gather(idx_va, c + 1, nabuf, ngsa).start()
                gather(idx_vb, c + 1, nbbuf, ngsb).start()
            gather(idx_va, c, abuf, gsa).wait()
            gather(idx_vb, c, bbuf, gsb).wait()
            @pl.when(c >= 2)
            def _():
                out_copy(c - 2, pbuf, osem).wait()
            pairup(abuf, bbuf, pbuf)
            out_copy(c, pbuf, osem).start()

        def pair(p, carry):
            step(2 * p, a0, gsa0, b0, gsb0, p0, os0, a1, gsa1, b1, gsb1)
            step(2 * p + 1, a1, gsa1, b1, gsb1, p1, os1, a0, gsa0, b0, gsb0)
            return carry

        lax.fori_loop(0, nch // 2, pair, 0)

        out_copy(nch - 2, p0, os0).wait()
        out_copy(nch - 1, p1, os1).wait()

    return k(table, idx2)


def _tc_finish(g2, pe_runs, w2, c2, lp, d, bsz):
    m = g2.shape[0]
    lhalf = m // bsz
    grid = lhalf // lp
    block_rows = lp * bsz
    scale = math.sqrt(d)

    def body(g_ref, pe_ref, w_ref, c_ref, o_ref):
        pe_blk = pe_ref[...][0]
        x3 = g_ref[...].reshape(lp, bsz, 2 * d) * scale + pe_blk[:, None, :]
        x = x3.reshape(block_rows, 2 * d)
        r = jnp.dot(x, w_ref[...], preferred_element_type=jnp.float32)
        z = x * (r + c_ref[...])
        z3 = z.reshape(lp, bsz, 2 * d)
        for j in range(lp):
            zj = z3[j]
            o_ref[0, j] = zj[:, :d].T
            o_ref[1, j] = zj[:, d:].T

    return pl.pallas_call(
        body,
        grid=(grid,),
        in_specs=[
            pl.BlockSpec((block_rows, 2 * d), lambda i: (i, 0)),
            pl.BlockSpec((1, lp, 2 * d), lambda i: (i, 0, 0)),
            pl.BlockSpec((2 * d, 2 * d), lambda i: (0, 0)),
            pl.BlockSpec((1, 2 * d), lambda i: (0, 0)),
        ],
        out_specs=pl.BlockSpec((2, lp, d, bsz), lambda i: (0, i, 0, 0)),
        out_shape=jax.ShapeDtypeStruct((2, lhalf, d, bsz), jnp.float32),
    )(g2, pe_runs, w2, c2)


def kernel(src, tok_embedding, W, b, pe):
    bsz, seqlen = src.shape
    d = tok_embedding.shape[1]
    n = bsz * seqlen

    idx2 = jnp.transpose(src).reshape(n // 128, 128)

    g = _sc_gather(tok_embedding, idx2)
    g2 = g.reshape(n // 2, 2 * d)

    lp = 4
    pe_runs = jnp.concatenate(
        [pe[0, : seqlen // 2], pe[0, seqlen // 2:]], axis=1)
    pe_runs = pe_runs.reshape(seqlen // 2 // lp, lp, 2 * d)

    wt = W.T
    w2 = jnp.zeros((2 * d, 2 * d), W.dtype)
    w2 = w2.at[:d, :d].set(wt).at[d:, d:].set(wt)
    c2 = jnp.tile(b + 1.0, 2)[None]

    z4 = _tc_finish(g2, pe_runs, w2, c2, lp, d, bsz)
    return jnp.transpose(z4.reshape(seqlen, d, bsz), (2, 0, 1))
